# Initial kernel scaffold; baseline (speedup 1.0000x reference)
import jax
import jax.numpy as jnp
from jax.experimental import pallas as pl


def kernel(x, edge_index, edge_weight, W1, b1, W2, b2, W3, b3, W4, b4, Wout, bout):
    raise NotImplementedError("write your pallas kernel here")


# speedup vs baseline: 2.5690x; 2.5690x over previous
import functools

import jax
import jax.numpy as jnp
from jax import lax
from jax.experimental import pallas as pl
from jax.experimental.pallas import tpu as pltpu
from jax.experimental.pallas import tpu_sc as plsc

N = 10000
NP = 10240
E = 160000
NW = 32
EPW = 5000
EPWP = 5120
ER = 40
F32 = jnp.float32
I32 = jnp.int32

_MESH = plsc.VectorSubcoreMesh(core_axis_name="c", subcore_axis_name="s")


def _dis_kernel(degp, xp):
    bm = 2048

    def body(degp_ref, x_ref, dis_ref, z0_ref):
        dp = degp_ref[...]
        d = dp[0, :, 0:1] + dp[1, :, 0:1]
        dis = jax.lax.rsqrt(d)
        dis_ref[...] = jnp.broadcast_to(dis, (bm, 128))
        z0_ref[...] = dis * x_ref[...]

    return pl.pallas_call(
        body,
        grid=(NP // bm,),
        in_specs=[
            pl.BlockSpec((2, bm, 128), lambda i: (0, i, 0)),
            pl.BlockSpec((bm, 128), lambda i: (i, 0)),
        ],
        out_specs=(
            pl.BlockSpec((bm, 128), lambda i: (i, 0)),
            pl.BlockSpec((bm, 128), lambda i: (i, 0)),
        ),
        out_shape=(
            jax.ShapeDtypeStruct((NP, 128), F32),
            jax.ShapeDtypeStruct((NP, 128), F32),
        ),
    )(degp, xp)


def _agg_kernel(Z, C, src2, dst2, ewr):
    Zflat = Z.reshape(NP * C, 128)
    edge_split = C == 1
    Z3 = Zflat if edge_split else Z.reshape(NP, C, 128)
    erows = ER if edge_split else 2 * ER
    QR = 8
    n_stages = erows // QR
    half = QR // 2
    out_ty = (
        jax.ShapeDtypeStruct((2, NP, 128), F32)
        if edge_split
        else jax.ShapeDtypeStruct((NP, C, 128), F32)
    )

    @functools.partial(
        pl.kernel,
        mesh=_MESH,
        out_type=out_ty,
        scratch_types=[
            pltpu.VMEM_SHARED((N, 128), F32),
            pltpu.VMEM((QR, 128), I32),
            pltpu.VMEM((64, 16), F32),
            pltpu.VMEM((QR, 128), I32),
            pltpu.VMEM((128, 128), F32),
            pltpu.VMEM((128, 128), F32),
            pltpu.SemaphoreType.DMA,
            pltpu.SemaphoreType.DMA,
        ],
    )
    def k(zf_h, z3_h, src_h, dst_h, ewr_h, out_h,
          acc_s, dst_v, ewr_v, gidx_v, rows_a, rows_b, sem_a, sem_b):
        cid = lax.axis_index("c")
        sid = lax.axis_index("s")
        ebase = (sid * 2 + cid) * ER if edge_split else sid * (2 * ER)
        node0 = sid * 624
        SUBS_A = [(0, 128), (128, 128), (256, 128), (384, 128), (512, 112)]
        SUBS_B = [(0, 128), (128, 128), (256, 128), (384, 128), (512, 128)]

        def scale_rows(buf, erow):
            for h in range(2):
                pltpu.sync_copy(
                    ewr_h.at[pl.ds(erow * 128 + h * 64, 64)], ewr_v
                )

                def sbody(r, _):
                    sc = ewr_v[r, :]
                    b = h * 64 + r
                    for i in range(8):
                        buf[b, pl.ds(16 * i, 16)] = (
                            buf[b, pl.ds(16 * i, 16)] * sc
                        )
                    return 0

                lax.fori_loop(0, 64, sbody, 0)

        def _zcopy(base, nr, c):
            if edge_split:
                pltpu.sync_copy(
                    z3_h.at[pl.ds(base, nr)], rows_a.at[pl.ds(0, nr)]
                )
            else:
                pltpu.sync_copy(
                    z3_h.at[pl.ds(base, nr), c], rows_a.at[pl.ds(0, nr)]
                )
            pltpu.sync_copy(rows_a.at[pl.ds(0, nr)], acc_s.at[pl.ds(base, nr)])

        def init_from_z(c):
            @pl.when(sid < 15)
            def _():
                for off, nr in SUBS_A:
                    _zcopy(node0 + off, nr, c)

            @pl.when(sid == 15)
            def _():
                for off, nr in SUBS_B:
                    _zcopy(9360 + off, nr, c)

        def init_zero():
            def zbody(r, _):
                for i in range(8):
                    rows_a[r, pl.ds(16 * i, 16)] = jnp.zeros((16,), F32)
                return 0

            lax.fori_loop(0, 128, zbody, 0)

            @pl.when(sid < 15)
            def _():
                for off, nr in SUBS_A:
                    pltpu.sync_copy(
                        rows_a.at[pl.ds(0, nr)], acc_s.at[pl.ds(node0 + off, nr)]
                    )

            @pl.when(sid == 15)
            def _():
                for off, nr in SUBS_B:
                    pltpu.sync_copy(
                        rows_a.at[pl.ds(0, nr)], acc_s.at[pl.ds(9360 + off, nr)]
                    )

        def chunk_pass(qbase):
            def start(row, buf, sem):
                pltpu.async_copy(zf_h.at[gidx_v.at[row]], buf, sem)

            def wait(row, buf, sem):
                pltpu.make_async_copy(zf_h.at[gidx_v.at[row]], buf, sem).wait()

            start(0, rows_a, sem_a)

            def body(i, _):
                b0 = 2 * i
                b1 = 2 * i + 1
                start(b1, rows_b, sem_b)
                wait(b0, rows_a, sem_a)
                scale_rows(rows_a, qbase + b0)
                pltpu.sync_copy(rows_a, acc_s.at[dst_v.at[b0]], add=True)

                @pl.when(i + 1 < half)
                def _():
                    start(b0 + 2, rows_a, sem_a)

                wait(b1, rows_b, sem_b)
                scale_rows(rows_b, qbase + b1)
                pltpu.sync_copy(rows_b, acc_s.at[dst_v.at[b1]], add=True)
                return 0

            lax.fori_loop(0, half, body, 0)

        def edge_sweep(c):
            def stage_body(q, _):
                qbase = ebase + q * QR
                pltpu.sync_copy(src_h.at[pl.ds(qbase, QR)], gidx_v)
                pltpu.sync_copy(dst_h.at[pl.ds(qbase, QR)], dst_v)
                if not edge_split:

                    def gbody(r, _):
                        for i in range(8):
                            gidx_v[r, pl.ds(16 * i, 16)] = (
                                gidx_v[r, pl.ds(16 * i, 16)] * C + c
                            )
                        return 0

                    lax.fori_loop(0, QR, gbody, 0)
                chunk_pass(qbase)
                return 0

            lax.fori_loop(0, n_stages, stage_body, 0)

        if edge_split:
            @pl.when(cid == 0)
            def _():
                init_from_z(0)

            @pl.when(cid == 1)
            def _():
                init_zero()

            plsc.subcore_barrier()
            edge_sweep(0)
            plsc.subcore_barrier()

            @pl.when(sid < 15)
            def _():
                for off, nr in SUBS_A:
                    pltpu.sync_copy(
                        acc_s.at[pl.ds(node0 + off, nr)],
                        out_h.at[cid, pl.ds(node0 + off, nr)],
                    )

            @pl.when(sid == 15)
            def _():
                for off, nr in SUBS_B:
                    pltpu.sync_copy(
                        acc_s.at[pl.ds(9360 + off, nr)],
                        out_h.at[cid, pl.ds(9360 + off, nr)],
                    )
        else:
            def chunk_body(t, _):
                c = 2 * t + cid
                init_from_z(c)
                plsc.subcore_barrier()
                edge_sweep(c)
                plsc.subcore_barrier()

                @pl.when(sid < 15)
                def _():
                    for off, nr in SUBS_A:
                        pltpu.sync_copy(
                            acc_s.at[pl.ds(node0 + off, nr)],
                            out_h.at[pl.ds(node0 + off, nr), c],
                        )

                @pl.when(sid == 15)
                def _():
                    for off, nr in SUBS_B:
                        pltpu.sync_copy(
                            acc_s.at[pl.ds(9360 + off, nr)],
                            out_h.at[pl.ds(9360 + off, nr), c],
                        )

                plsc.subcore_barrier()
                return 0

            lax.fori_loop(0, C // 2, chunk_body, 0)

    return k(Zflat, Z3, src2, dst2, ewr)


def _matmul(x, w, bias=None, pre_bias=None, pre_dis=None, post_dis=None,
            post_relu=False, x2=None, bm=2048, bk=512, bn=1024):
    M, K = x.shape
    Kw, Nn = w.shape
    bk = min(bk, K)
    bn = min(bn, Nn)
    grid = (M // bm, Nn // bn, K // bk)
    nk = grid[2]

    def body(*refs):
        it = iter(refs)
        x_ref = next(it)
        x2_ref = next(it) if x2 is not None else None
        pd_ref = next(it) if pre_dis is not None else None
        pb_ref = next(it) if pre_bias is not None else None
        w_ref = next(it)
        b_ref = next(it) if bias is not None else None
        od_ref = next(it) if post_dis is not None else None
        o_ref = next(it)
        acc_ref = next(it)
        kk = pl.program_id(2)

        @pl.when(kk == 0)
        def _():
            acc_ref[...] = jnp.zeros_like(acc_ref)

        xv = x_ref[...]
        if x2_ref is not None:
            xv = xv + x2_ref[...]
        if pd_ref is not None:
            xv = xv * pd_ref[...][:, 0:1]
        if pb_ref is not None:
            xv = jax.nn.relu(xv + pb_ref[...])
        acc_ref[...] += jnp.dot(xv, w_ref[...], preferred_element_type=F32)

        @pl.when(kk == nk - 1)
        def _():
            r = acc_ref[...]
            if b_ref is not None:
                r = r + b_ref[...]
            if post_relu:
                r = jax.nn.relu(r)
            if od_ref is not None:
                r = r * od_ref[...][:, 0:1]
            o_ref[...] = r

    in_specs = [pl.BlockSpec((bm, bk), lambda i, j, kq: (i, kq))]
    args = [x]
    if x2 is not None:
        in_specs.append(pl.BlockSpec((bm, bk), lambda i, j, kq: (i, kq)))
        args.append(x2)
    if pre_dis is not None:
        in_specs.append(pl.BlockSpec((bm, 128), lambda i, j, kq: (i, 0)))
        args.append(pre_dis)
    if pre_bias is not None:
        in_specs.append(pl.BlockSpec((1, bk), lambda i, j, kq: (0, kq)))
        args.append(pre_bias.reshape(1, K))
    in_specs.append(pl.BlockSpec((bk, bn), lambda i, j, kq: (kq, j)))
    args.append(w)
    if bias is not None:
        in_specs.append(pl.BlockSpec((1, bn), lambda i, j, kq: (0, j)))
        args.append(bias.reshape(1, Nn))
    if post_dis is not None:
        in_specs.append(pl.BlockSpec((bm, 128), lambda i, j, kq: (i, 0)))
        args.append(post_dis)

    return pl.pallas_call(
        body,
        grid=grid,
        in_specs=in_specs,
        out_specs=pl.BlockSpec((bm, bn), lambda i, j, kq: (i, j)),
        out_shape=jax.ShapeDtypeStruct((M, Nn), F32),
        scratch_shapes=[pltpu.VMEM((bm, bn), F32)],
        compiler_params=pltpu.CompilerParams(
            dimension_semantics=("parallel", "parallel", "arbitrary")
        ),
    )(*args)


def kernel(x, edge_index, edge_weight, W1, b1, W2, b2, W3, b3, W4, b4,
           Wout, bout):
    src = edge_index[0].astype(I32)
    dst = edge_index[1].astype(I32)
    ew = edge_weight.astype(F32)

    def e2d(a):
        return jnp.pad(
            a.reshape(NW, EPW), ((0, 0), (0, EPWP - EPW))
        ).reshape(NW * ER, 128)

    src2, dst2, ew2 = e2d(src), e2d(dst), e2d(ew)
    ewr = jnp.broadcast_to(ew2.reshape(NW * ER * 128, 1), (NW * ER * 128, 16))
    xp = jnp.pad(x, ((0, NP - N), (0, 0)))

    agg = functools.partial(_agg_kernel, src2=src2, dst2=dst2, ewr=ewr)

    degp = agg(jnp.ones((NP, 128), F32), 1)
    dis, z0 = _dis_kernel(degp, xp)

    s0 = agg(z0, 1)
    z1 = _matmul(s0[0], W1, bias=b1, post_relu=True, x2=s0[1],
                 pre_dis=dis, post_dis=dis, bk=128)
    s1 = agg(z1, 8).reshape(NP, 1024)
    z2 = _matmul(s1, W2, bias=b2, post_relu=True,
                 pre_dis=dis, post_dis=dis)
    s2 = agg(z2, 16).reshape(NP, 2048)
    h3 = _matmul(s2, W3, bias=b3, post_relu=True, pre_dis=dis)
    z4 = _matmul(h3, W4, post_dis=dis)
    s4 = agg(z4, 16).reshape(NP, 2048)
    wop = jnp.pad(Wout, ((0, 0), (0, 128 - Wout.shape[1])))
    bop = jnp.pad(bout, (0, 128 - bout.shape[0]))
    out = _matmul(s4, wop, bias=bop, pre_bias=b4, pre_dis=dis, bn=128)
    return out[:N, : Wout.shape[1]]


# speedup vs baseline: 2.7058x; 1.0532x over previous
import functools

import jax
import jax.numpy as jnp
from jax import lax
from jax.experimental import pallas as pl
from jax.experimental.pallas import tpu as pltpu
from jax.experimental.pallas import tpu_sc as plsc

N = 10000
NP = 10240
E = 160000
NW = 32
EPW = 5000
EPWP = 5120
ER = 40
F32 = jnp.float32
I32 = jnp.int32

_MESH = plsc.VectorSubcoreMesh(core_axis_name="c", subcore_axis_name="s")


def _dis_kernel(degp, xp):
    bm = 2048

    def body(degp_ref, x_ref, dis_ref, z0_ref):
        dp = degp_ref[...]
        d = dp[0, :, 0:1] + dp[1, :, 0:1]
        dis = jax.lax.rsqrt(d)
        dis_ref[...] = jnp.broadcast_to(dis, (bm, 128))
        z0_ref[...] = dis * x_ref[...]

    return pl.pallas_call(
        body,
        grid=(NP // bm,),
        in_specs=[
            pl.BlockSpec((2, bm, 128), lambda i: (0, i, 0)),
            pl.BlockSpec((bm, 128), lambda i: (i, 0)),
        ],
        out_specs=(
            pl.BlockSpec((bm, 128), lambda i: (i, 0)),
            pl.BlockSpec((bm, 128), lambda i: (i, 0)),
        ),
        out_shape=(
            jax.ShapeDtypeStruct((NP, 128), F32),
            jax.ShapeDtypeStruct((NP, 128), F32),
        ),
    )(degp, xp)


def _agg_kernel(Z, C, src2, dst2, ewr):
    Zflat = Z.reshape(NP * C, 128)
    edge_split = C == 1
    Z3 = Zflat if edge_split else Z.reshape(NP, C, 128)
    erows = ER if edge_split else 2 * ER
    QR = 8
    n_stages = erows // QR
    half = QR // 2
    out_ty = (
        jax.ShapeDtypeStruct((2, NP, 128), F32)
        if edge_split
        else jax.ShapeDtypeStruct((NP, C, 128), F32)
    )

    @functools.partial(
        pl.kernel,
        mesh=_MESH,
        out_type=out_ty,
        scratch_types=[
            pltpu.VMEM_SHARED((N, 128), F32),
            pltpu.VMEM((QR, 128), I32),
            pltpu.VMEM((64, 16), F32),
            pltpu.VMEM((QR, 128), I32),
            pltpu.VMEM((128, 128), F32),
            pltpu.VMEM((128, 128), F32),
            pltpu.SemaphoreType.DMA,
            pltpu.SemaphoreType.DMA,
            pltpu.SemaphoreType.DMA,
            pltpu.SemaphoreType.DMA,
        ],
    )
    def k(zf_h, z3_h, src_h, dst_h, ewr_h, out_h,
          acc_s, dst_v, ewr_v, gidx_v, rows_a, rows_b,
          sem_a, sem_b, sem_sa, sem_sb):
        cid = lax.axis_index("c")
        sid = lax.axis_index("s")
        ebase = (sid * 2 + cid) * ER if edge_split else sid * (2 * ER)
        node0 = sid * 624
        SUBS_A = [(0, 128), (128, 128), (256, 128), (384, 128), (512, 112)]
        SUBS_B = [(0, 128), (128, 128), (256, 128), (384, 128), (512, 128)]

        def scale_rows(buf, erow):
            for h in range(2):
                pltpu.sync_copy(
                    ewr_h.at[pl.ds(erow * 128 + h * 64, 64)], ewr_v
                )

                def sbody(r, _):
                    sc = ewr_v[r, :]
                    b = h * 64 + r
                    for i in range(8):
                        buf[b, pl.ds(16 * i, 16)] = (
                            buf[b, pl.ds(16 * i, 16)] * sc
                        )
                    return 0

                lax.fori_loop(0, 64, sbody, 0)

        def _zcopy(base, nr, c):
            if edge_split:
                pltpu.sync_copy(
                    z3_h.at[pl.ds(base, nr)], rows_a.at[pl.ds(0, nr)]
                )
            else:
                pltpu.sync_copy(
                    z3_h.at[pl.ds(base, nr), c], rows_a.at[pl.ds(0, nr)]
                )
            pltpu.sync_copy(rows_a.at[pl.ds(0, nr)], acc_s.at[pl.ds(base, nr)])

        def init_from_z(c):
            @pl.when(sid < 15)
            def _():
                for off, nr in SUBS_A:
                    _zcopy(node0 + off, nr, c)

            @pl.when(sid == 15)
            def _():
                for off, nr in SUBS_B:
                    _zcopy(9360 + off, nr, c)

        def init_zero():
            def zbody(r, _):
                for i in range(8):
                    rows_a[r, pl.ds(16 * i, 16)] = jnp.zeros((16,), F32)
                return 0

            lax.fori_loop(0, 128, zbody, 0)

            @pl.when(sid < 15)
            def _():
                for off, nr in SUBS_A:
                    pltpu.sync_copy(
                        rows_a.at[pl.ds(0, nr)], acc_s.at[pl.ds(node0 + off, nr)]
                    )

            @pl.when(sid == 15)
            def _():
                for off, nr in SUBS_B:
                    pltpu.sync_copy(
                        rows_a.at[pl.ds(0, nr)], acc_s.at[pl.ds(9360 + off, nr)]
                    )

        def chunk_pass(qbase):
            def start_g(row, buf, sem):
                pltpu.async_copy(zf_h.at[gidx_v.at[row]], buf, sem)

            def wait_g(row, buf, sem):
                pltpu.make_async_copy(zf_h.at[gidx_v.at[row]], buf, sem).wait()

            def start_s(row, buf, sem):
                pltpu.async_copy(buf, acc_s.at[dst_v.at[row]], sem, add=True)

            def wait_s(row, buf, sem):
                pltpu.make_async_copy(buf, acc_s.at[dst_v.at[row]], sem).wait()

            start_g(0, rows_a, sem_a)
            start_g(1, rows_b, sem_b)

            def body(i, _):
                b0 = 2 * i
                b1 = 2 * i + 1
                wait_g(b0, rows_a, sem_a)
                scale_rows(rows_a, qbase + b0)
                start_s(b0, rows_a, sem_sa)
                wait_g(b1, rows_b, sem_b)
                scale_rows(rows_b, qbase + b1)
                start_s(b1, rows_b, sem_sb)

                @pl.when(i + 1 < half)
                def _():
                    wait_s(b0, rows_a, sem_sa)
                    start_g(b0 + 2, rows_a, sem_a)
                    wait_s(b1, rows_b, sem_sb)
                    start_g(b1 + 2, rows_b, sem_b)
                return 0

            lax.fori_loop(0, half, body, 0)
            wait_s(QR - 2, rows_a, sem_sa)
            wait_s(QR - 1, rows_b, sem_sb)

        def edge_sweep(c):
            def stage_body(q, _):
                qbase = ebase + q * QR
                pltpu.sync_copy(src_h.at[pl.ds(qbase, QR)], gidx_v)
                pltpu.sync_copy(dst_h.at[pl.ds(qbase, QR)], dst_v)
                if not edge_split:

                    def gbody(r, _):
                        for i in range(8):
                            gidx_v[r, pl.ds(16 * i, 16)] = (
                                gidx_v[r, pl.ds(16 * i, 16)] * C + c
                            )
                        return 0

                    lax.fori_loop(0, QR, gbody, 0)
                chunk_pass(qbase)
                return 0

            lax.fori_loop(0, n_stages, stage_body, 0)

        if edge_split:
            @pl.when(cid == 0)
            def _():
                init_from_z(0)

            @pl.when(cid == 1)
            def _():
                init_zero()

            plsc.subcore_barrier()
            edge_sweep(0)
            plsc.subcore_barrier()

            @pl.when(sid < 15)
            def _():
                for off, nr in SUBS_A:
                    pltpu.sync_copy(
                        acc_s.at[pl.ds(node0 + off, nr)],
                        out_h.at[cid, pl.ds(node0 + off, nr)],
                    )

            @pl.when(sid == 15)
            def _():
                for off, nr in SUBS_B:
                    pltpu.sync_copy(
                        acc_s.at[pl.ds(9360 + off, nr)],
                        out_h.at[cid, pl.ds(9360 + off, nr)],
                    )
        else:
            def chunk_body(t, _):
                c = 2 * t + cid
                init_from_z(c)
                plsc.subcore_barrier()
                edge_sweep(c)
                plsc.subcore_barrier()

                @pl.when(sid < 15)
                def _():
                    for off, nr in SUBS_A:
                        pltpu.sync_copy(
                            acc_s.at[pl.ds(node0 + off, nr)],
                            out_h.at[pl.ds(node0 + off, nr), c],
                        )

                @pl.when(sid == 15)
                def _():
                    for off, nr in SUBS_B:
                        pltpu.sync_copy(
                            acc_s.at[pl.ds(9360 + off, nr)],
                            out_h.at[pl.ds(9360 + off, nr), c],
                        )

                plsc.subcore_barrier()
                return 0

            lax.fori_loop(0, C // 2, chunk_body, 0)

    return k(Zflat, Z3, src2, dst2, ewr)


def _matmul(x, w, bias=None, pre_bias=None, pre_dis=None, post_dis=None,
            post_relu=False, x2=None, bm=2048, bk=512, bn=1024):
    M, K = x.shape
    Kw, Nn = w.shape
    bk = min(bk, K)
    bn = min(bn, Nn)
    grid = (M // bm, Nn // bn, K // bk)
    nk = grid[2]

    def body(*refs):
        it = iter(refs)
        x_ref = next(it)
        x2_ref = next(it) if x2 is not None else None
        pd_ref = next(it) if pre_dis is not None else None
        pb_ref = next(it) if pre_bias is not None else None
        w_ref = next(it)
        b_ref = next(it) if bias is not None else None
        od_ref = next(it) if post_dis is not None else None
        o_ref = next(it)
        acc_ref = next(it)
        kk = pl.program_id(2)

        @pl.when(kk == 0)
        def _():
            acc_ref[...] = jnp.zeros_like(acc_ref)

        xv = x_ref[...]
        if x2_ref is not None:
            xv = xv + x2_ref[...]
        if pd_ref is not None:
            xv = xv * pd_ref[...][:, 0:1]
        if pb_ref is not None:
            xv = jax.nn.relu(xv + pb_ref[...])
        acc_ref[...] += jnp.dot(xv, w_ref[...], preferred_element_type=F32)

        @pl.when(kk == nk - 1)
        def _():
            r = acc_ref[...]
            if b_ref is not None:
                r = r + b_ref[...]
            if post_relu:
                r = jax.nn.relu(r)
            if od_ref is not None:
                r = r * od_ref[...][:, 0:1]
            o_ref[...] = r

    in_specs = [pl.BlockSpec((bm, bk), lambda i, j, kq: (i, kq))]
    args = [x]
    if x2 is not None:
        in_specs.append(pl.BlockSpec((bm, bk), lambda i, j, kq: (i, kq)))
        args.append(x2)
    if pre_dis is not None:
        in_specs.append(pl.BlockSpec((bm, 128), lambda i, j, kq: (i, 0)))
        args.append(pre_dis)
    if pre_bias is not None:
        in_specs.append(pl.BlockSpec((1, bk), lambda i, j, kq: (0, kq)))
        args.append(pre_bias.reshape(1, K))
    in_specs.append(pl.BlockSpec((bk, bn), lambda i, j, kq: (kq, j)))
    args.append(w)
    if bias is not None:
        in_specs.append(pl.BlockSpec((1, bn), lambda i, j, kq: (0, j)))
        args.append(bias.reshape(1, Nn))
    if post_dis is not None:
        in_specs.append(pl.BlockSpec((bm, 128), lambda i, j, kq: (i, 0)))
        args.append(post_dis)

    return pl.pallas_call(
        body,
        grid=grid,
        in_specs=in_specs,
        out_specs=pl.BlockSpec((bm, bn), lambda i, j, kq: (i, j)),
        out_shape=jax.ShapeDtypeStruct((M, Nn), F32),
        scratch_shapes=[pltpu.VMEM((bm, bn), F32)],
        compiler_params=pltpu.CompilerParams(
            dimension_semantics=("parallel", "parallel", "arbitrary")
        ),
    )(*args)


def kernel(x, edge_index, edge_weight, W1, b1, W2, b2, W3, b3, W4, b4,
           Wout, bout):
    src = edge_index[0].astype(I32)
    dst = edge_index[1].astype(I32)
    ew = edge_weight.astype(F32)

    def e2d(a):
        return jnp.pad(
            a.reshape(NW, EPW), ((0, 0), (0, EPWP - EPW))
        ).reshape(NW * ER, 128)

    src2, dst2, ew2 = e2d(src), e2d(dst), e2d(ew)
    ewr = jnp.broadcast_to(ew2.reshape(NW * ER * 128, 1), (NW * ER * 128, 16))
    xp = jnp.pad(x, ((0, NP - N), (0, 0)))

    agg = functools.partial(_agg_kernel, src2=src2, dst2=dst2, ewr=ewr)

    degp = agg(jnp.ones((NP, 128), F32), 1)
    dis, z0 = _dis_kernel(degp, xp)

    s0 = agg(z0, 1)
    z1 = _matmul(s0[0], W1, bias=b1, post_relu=True, x2=s0[1],
                 pre_dis=dis, post_dis=dis, bk=128)
    s1 = agg(z1, 8).reshape(NP, 1024)
    z2 = _matmul(s1, W2, bias=b2, post_relu=True,
                 pre_dis=dis, post_dis=dis)
    s2 = agg(z2, 16).reshape(NP, 2048)
    h3 = _matmul(s2, W3, bias=b3, post_relu=True, pre_dis=dis)
    z4 = _matmul(h3, W4, post_dis=dis)
    s4 = agg(z4, 16).reshape(NP, 2048)
    wop = jnp.pad(Wout, ((0, 0), (0, 128 - Wout.shape[1])))
    bop = jnp.pad(bout, (0, 128 - bout.shape[0]))
    out = _matmul(s4, wop, bias=bop, pre_bias=b4, pre_dis=dis, bn=128)
    return out[:N, : Wout.shape[1]]


# speedup vs baseline: 2.7641x; 1.0215x over previous
import functools

import jax
import jax.numpy as jnp
from jax import lax
from jax.experimental import pallas as pl
from jax.experimental.pallas import tpu as pltpu
from jax.experimental.pallas import tpu_sc as plsc

N = 10000
NP = 10240
E = 160000
NW = 32
EPW = 5000
EPWP = 5120
ER = 40
F32 = jnp.float32
I32 = jnp.int32

_MESH = plsc.VectorSubcoreMesh(core_axis_name="c", subcore_axis_name="s")


def _dis_kernel(degp, xp):
    bm = 2048

    def body(degp_ref, x_ref, dis_ref, z0_ref):
        dp = degp_ref[...]
        d = dp[0, :, 0:1] + dp[1, :, 0:1]
        dis = jax.lax.rsqrt(d)
        dis_ref[...] = jnp.broadcast_to(dis, (bm, 128))
        z0_ref[...] = dis * x_ref[...]

    return pl.pallas_call(
        body,
        grid=(NP // bm,),
        in_specs=[
            pl.BlockSpec((2, bm, 128), lambda i: (0, i, 0)),
            pl.BlockSpec((bm, 128), lambda i: (i, 0)),
        ],
        out_specs=(
            pl.BlockSpec((bm, 128), lambda i: (i, 0)),
            pl.BlockSpec((bm, 128), lambda i: (i, 0)),
        ),
        out_shape=(
            jax.ShapeDtypeStruct((NP, 128), F32),
            jax.ShapeDtypeStruct((NP, 128), F32),
        ),
    )(degp, xp)


def _agg_kernel(Z, C, src2, dst2, ewr):
    Zflat = Z.reshape(NP * C, 128)
    edge_split = C == 1
    Z3 = Zflat if edge_split else Z.reshape(NP, C, 128)
    erows = ER if edge_split else 2 * ER
    QR = 8
    n_stages = erows // QR
    half = QR // 2
    out_ty = (
        jax.ShapeDtypeStruct((2, NP, 128), F32)
        if edge_split
        else jax.ShapeDtypeStruct((NP, C, 128), F32)
    )

    @functools.partial(
        pl.kernel,
        mesh=_MESH,
        out_type=out_ty,
        scratch_types=[
            pltpu.VMEM_SHARED((N, 128), F32),
            pltpu.VMEM((QR, 128), I32),
            pltpu.VMEM((64, 16), F32),
            pltpu.VMEM((QR, 128), I32),
            pltpu.VMEM((128, 128), F32),
            pltpu.VMEM((128, 128), F32),
            pltpu.SemaphoreType.DMA,
            pltpu.SemaphoreType.DMA,
            pltpu.SemaphoreType.DMA,
            pltpu.SemaphoreType.DMA,
            pltpu.SemaphoreType.DMA,
        ],
    )
    def k(zf_h, z3_h, src_h, dst_h, ewr_h, out_h,
          acc_s, dst_v, ewr_v, gidx_v, rows_a, rows_b,
          sem_a, sem_b, sem_sa, sem_sb, sem_e):
        cid = lax.axis_index("c")
        sid = lax.axis_index("s")
        ebase = (sid * 2 + cid) * ER if edge_split else sid * (2 * ER)
        node0 = sid * 624
        SUBS_A = [(0, 128), (128, 128), (256, 128), (384, 128), (512, 112)]
        SUBS_B = [(0, 128), (128, 128), (256, 128), (384, 128), (512, 128)]

        def ewr_fetch(erow, h):
            pltpu.async_copy(
                ewr_h.at[pl.ds(erow * 128 + h * 64, 64)], ewr_v, sem_e
            )

        def ewr_wait(erow, h):
            pltpu.make_async_copy(
                ewr_h.at[pl.ds(erow * 128 + h * 64, 64)], ewr_v, sem_e
            ).wait()

        def scale_rows(buf, erow):
            for h in range(2):
                ewr_wait(erow, h)

                def sbody(r, _):
                    sc = ewr_v[r, :]
                    b = h * 64 + r
                    for i in range(8):
                        buf[b, pl.ds(16 * i, 16)] = (
                            buf[b, pl.ds(16 * i, 16)] * sc
                        )
                    return 0

                lax.fori_loop(0, 64, sbody, 0)
                if h == 0:
                    ewr_fetch(erow, 1)

        def _zcopy(base, nr, c):
            if edge_split:
                pltpu.sync_copy(
                    z3_h.at[pl.ds(base, nr)], rows_a.at[pl.ds(0, nr)]
                )
            else:
                pltpu.sync_copy(
                    z3_h.at[pl.ds(base, nr), c], rows_a.at[pl.ds(0, nr)]
                )
            pltpu.sync_copy(rows_a.at[pl.ds(0, nr)], acc_s.at[pl.ds(base, nr)])

        def init_from_z(c):
            @pl.when(sid < 15)
            def _():
                for off, nr in SUBS_A:
                    _zcopy(node0 + off, nr, c)

            @pl.when(sid == 15)
            def _():
                for off, nr in SUBS_B:
                    _zcopy(9360 + off, nr, c)

        def init_zero():
            def zbody(r, _):
                for i in range(8):
                    rows_a[r, pl.ds(16 * i, 16)] = jnp.zeros((16,), F32)
                return 0

            lax.fori_loop(0, 128, zbody, 0)

            @pl.when(sid < 15)
            def _():
                for off, nr in SUBS_A:
                    pltpu.sync_copy(
                        rows_a.at[pl.ds(0, nr)], acc_s.at[pl.ds(node0 + off, nr)]
                    )

            @pl.when(sid == 15)
            def _():
                for off, nr in SUBS_B:
                    pltpu.sync_copy(
                        rows_a.at[pl.ds(0, nr)], acc_s.at[pl.ds(9360 + off, nr)]
                    )

        def chunk_pass(qbase):
            def start_g(row, buf, sem):
                pltpu.async_copy(zf_h.at[gidx_v.at[row]], buf, sem)

            def wait_g(row, buf, sem):
                pltpu.make_async_copy(zf_h.at[gidx_v.at[row]], buf, sem).wait()

            def start_s(row, buf, sem):
                pltpu.async_copy(buf, acc_s.at[dst_v.at[row]], sem, add=True)

            def wait_s(row, buf, sem):
                pltpu.make_async_copy(buf, acc_s.at[dst_v.at[row]], sem).wait()

            start_g(0, rows_a, sem_a)
            start_g(1, rows_b, sem_b)
            ewr_fetch(qbase, 0)

            def body(i, _):
                b0 = 2 * i
                b1 = 2 * i + 1
                wait_g(b0, rows_a, sem_a)
                scale_rows(rows_a, qbase + b0)
                ewr_fetch(qbase + b1, 0)
                start_s(b0, rows_a, sem_sa)
                wait_g(b1, rows_b, sem_b)
                scale_rows(rows_b, qbase + b1)

                @pl.when(i + 1 < half)
                def _():
                    ewr_fetch(qbase + b0 + 2, 0)

                start_s(b1, rows_b, sem_sb)

                @pl.when(i + 1 < half)
                def _():
                    wait_s(b0, rows_a, sem_sa)
                    start_g(b0 + 2, rows_a, sem_a)
                    wait_s(b1, rows_b, sem_sb)
                    start_g(b1 + 2, rows_b, sem_b)
                return 0

            lax.fori_loop(0, half, body, 0)
            wait_s(QR - 2, rows_a, sem_sa)
            wait_s(QR - 1, rows_b, sem_sb)

        def edge_sweep(c):
            def stage_body(q, _):
                qbase = ebase + q * QR
                pltpu.sync_copy(src_h.at[pl.ds(qbase, QR)], gidx_v)
                pltpu.sync_copy(dst_h.at[pl.ds(qbase, QR)], dst_v)
                if not edge_split:

                    def gbody(r, _):
                        for i in range(8):
                            gidx_v[r, pl.ds(16 * i, 16)] = (
                                gidx_v[r, pl.ds(16 * i, 16)] * C + c
                            )
                        return 0

                    lax.fori_loop(0, QR, gbody, 0)
                chunk_pass(qbase)
                return 0

            lax.fori_loop(0, n_stages, stage_body, 0)

        if edge_split:
            @pl.when(cid == 0)
            def _():
                init_from_z(0)

            @pl.when(cid == 1)
            def _():
                init_zero()

            plsc.subcore_barrier()
            edge_sweep(0)
            plsc.subcore_barrier()

            @pl.when(sid < 15)
            def _():
                for off, nr in SUBS_A:
                    pltpu.sync_copy(
                        acc_s.at[pl.ds(node0 + off, nr)],
                        out_h.at[cid, pl.ds(node0 + off, nr)],
                    )

            @pl.when(sid == 15)
            def _():
                for off, nr in SUBS_B:
                    pltpu.sync_copy(
                        acc_s.at[pl.ds(9360 + off, nr)],
                        out_h.at[cid, pl.ds(9360 + off, nr)],
                    )
        else:
            def chunk_body(t, _):
                c = 2 * t + cid
                init_from_z(c)
                plsc.subcore_barrier()
                edge_sweep(c)
                plsc.subcore_barrier()

                @pl.when(sid < 15)
                def _():
                    for off, nr in SUBS_A:
                        pltpu.sync_copy(
                            acc_s.at[pl.ds(node0 + off, nr)],
                            out_h.at[pl.ds(node0 + off, nr), c],
                        )

                @pl.when(sid == 15)
                def _():
                    for off, nr in SUBS_B:
                        pltpu.sync_copy(
                            acc_s.at[pl.ds(9360 + off, nr)],
                            out_h.at[pl.ds(9360 + off, nr), c],
                        )

                plsc.subcore_barrier()
                return 0

            lax.fori_loop(0, C // 2, chunk_body, 0)

    return k(Zflat, Z3, src2, dst2, ewr)


def _matmul(x, w, bias=None, pre_bias=None, pre_dis=None, post_dis=None,
            post_relu=False, x2=None, bm=2048, bk=512, bn=1024):
    M, K = x.shape
    Kw, Nn = w.shape
    bk = min(bk, K)
    bn = min(bn, Nn)
    grid = (M // bm, Nn // bn, K // bk)
    nk = grid[2]

    def body(*refs):
        it = iter(refs)
        x_ref = next(it)
        x2_ref = next(it) if x2 is not None else None
        pd_ref = next(it) if pre_dis is not None else None
        pb_ref = next(it) if pre_bias is not None else None
        w_ref = next(it)
        b_ref = next(it) if bias is not None else None
        od_ref = next(it) if post_dis is not None else None
        o_ref = next(it)
        acc_ref = next(it)
        kk = pl.program_id(2)

        @pl.when(kk == 0)
        def _():
            acc_ref[...] = jnp.zeros_like(acc_ref)

        xv = x_ref[...]
        if x2_ref is not None:
            xv = xv + x2_ref[...]
        if pd_ref is not None:
            xv = xv * pd_ref[...][:, 0:1]
        if pb_ref is not None:
            xv = jax.nn.relu(xv + pb_ref[...])
        acc_ref[...] += jnp.dot(xv, w_ref[...], preferred_element_type=F32)

        @pl.when(kk == nk - 1)
        def _():
            r = acc_ref[...]
            if b_ref is not None:
                r = r + b_ref[...]
            if post_relu:
                r = jax.nn.relu(r)
            if od_ref is not None:
                r = r * od_ref[...][:, 0:1]
            o_ref[...] = r

    in_specs = [pl.BlockSpec((bm, bk), lambda i, j, kq: (i, kq))]
    args = [x]
    if x2 is not None:
        in_specs.append(pl.BlockSpec((bm, bk), lambda i, j, kq: (i, kq)))
        args.append(x2)
    if pre_dis is not None:
        in_specs.append(pl.BlockSpec((bm, 128), lambda i, j, kq: (i, 0)))
        args.append(pre_dis)
    if pre_bias is not None:
        in_specs.append(pl.BlockSpec((1, bk), lambda i, j, kq: (0, kq)))
        args.append(pre_bias.reshape(1, K))
    in_specs.append(pl.BlockSpec((bk, bn), lambda i, j, kq: (kq, j)))
    args.append(w)
    if bias is not None:
        in_specs.append(pl.BlockSpec((1, bn), lambda i, j, kq: (0, j)))
        args.append(bias.reshape(1, Nn))
    if post_dis is not None:
        in_specs.append(pl.BlockSpec((bm, 128), lambda i, j, kq: (i, 0)))
        args.append(post_dis)

    return pl.pallas_call(
        body,
        grid=grid,
        in_specs=in_specs,
        out_specs=pl.BlockSpec((bm, bn), lambda i, j, kq: (i, j)),
        out_shape=jax.ShapeDtypeStruct((M, Nn), F32),
        scratch_shapes=[pltpu.VMEM((bm, bn), F32)],
        compiler_params=pltpu.CompilerParams(
            dimension_semantics=("parallel", "parallel", "arbitrary")
        ),
    )(*args)


def kernel(x, edge_index, edge_weight, W1, b1, W2, b2, W3, b3, W4, b4,
           Wout, bout):
    src = edge_index[0].astype(I32)
    dst = edge_index[1].astype(I32)
    ew = edge_weight.astype(F32)

    def e2d(a):
        return jnp.pad(
            a.reshape(NW, EPW), ((0, 0), (0, EPWP - EPW))
        ).reshape(NW * ER, 128)

    src2, dst2, ew2 = e2d(src), e2d(dst), e2d(ew)
    ewr = jnp.broadcast_to(ew2.reshape(NW * ER * 128, 1), (NW * ER * 128, 16))
    xp = jnp.pad(x, ((0, NP - N), (0, 0)))

    agg = functools.partial(_agg_kernel, src2=src2, dst2=dst2, ewr=ewr)

    degp = agg(jnp.ones((NP, 128), F32), 1)
    dis, z0 = _dis_kernel(degp, xp)

    s0 = agg(z0, 1)
    z1 = _matmul(s0[0], W1, bias=b1, post_relu=True, x2=s0[1],
                 pre_dis=dis, post_dis=dis, bk=128)
    s1 = agg(z1, 8).reshape(NP, 1024)
    z2 = _matmul(s1, W2, bias=b2, post_relu=True,
                 pre_dis=dis, post_dis=dis)
    s2 = agg(z2, 16).reshape(NP, 2048)
    h3 = _matmul(s2, W3, bias=b3, post_relu=True, pre_dis=dis)
    z4 = _matmul(h3, W4, post_dis=dis)
    s4 = agg(z4, 16).reshape(NP, 2048)
    wop = jnp.pad(Wout, ((0, 0), (0, 128 - Wout.shape[1])))
    bop = jnp.pad(bout, (0, 128 - bout.shape[0]))
    out = _matmul(s4, wop, bias=bop, pre_bias=b4, pre_dis=dis, bn=128)
    return out[:N, : Wout.shape[1]]
